# baseline (speedup 1.0000x reference)
import functools

import jax
import jax.numpy as jnp
from jax import lax
from jax.experimental import pallas as pl
from jax.experimental.pallas import tpu as pltpu
from jax.experimental.pallas import tpu_sc as plsc

_OMEGA = 64
_SINK = 5
_KWIN = 3
_PRATIO = 0.2

_S_TC = 1280
_RCH = 8
_LANES = 16


def _tc_partial_body(attn_ref, out_ref):
    out_ref[...] = jnp.sum(attn_ref[0], axis=0, keepdims=True)[None]


def _make_tc_partial(H, s_tc, cpad):
    return pl.pallas_call(
        _tc_partial_body,
        grid=(H,),
        in_specs=[pl.BlockSpec((1, s_tc, cpad), lambda h: (h, 0, 0))],
        out_specs=pl.BlockSpec((1, 1, cpad), lambda h: (h, 0, 0)),
        out_shape=jax.ShapeDtypeStruct((H, 1, cpad), jnp.float32),
    )


def _make_sc_partial(H, S, s_tc, cpad):
    rows_per_tile = (S - s_tc) // 2
    nch = rows_per_tile // _RCH
    assert nch % 2 == 0
    ncs = cpad // _LANES
    mesh = plsc.VectorSubcoreMesh(core_axis_name="c", subcore_axis_name="s")

    @functools.partial(
        pl.kernel,
        mesh=mesh,
        out_type=jax.ShapeDtypeStruct((2, H, cpad), jnp.float32),
        scratch_types=[pltpu.VMEM((cpad,), jnp.float32),
                       pltpu.VMEM((_RCH, cpad), jnp.float32),
                       pltpu.VMEM((_RCH, cpad), jnp.float32),
                       pltpu.SemaphoreType.DMA,
                       pltpu.SemaphoreType.DMA],
    )
    def _sc_partial(attn_hbm, out_hbm, acc, buf0, buf1, sem0, sem1):
        h = lax.axis_index("s")
        half = lax.axis_index("c")
        r0 = s_tc + half * rows_per_tile

        zero = jnp.zeros((_LANES,), jnp.float32)
        for c in range(ncs):
            acc[pl.ds(c * _LANES, _LANES)] = zero

        def chunk_src(ch):
            return attn_hbm.at[h, pl.ds(r0 + ch * _RCH, _RCH),
                               pl.ds(0, cpad)]

        def accumulate(buf):
            for c in range(ncs):
                sl = pl.ds(c * _LANES, _LANES)
                v0 = buf[0, sl]
                v1 = buf[1, sl]
                for r in range(2, _RCH, 2):
                    v0 = v0 + buf[r, sl]
                    v1 = v1 + buf[r + 1, sl]
                acc[sl] = acc[sl] + (v0 + v1)

        pltpu.async_copy(chunk_src(0), buf0, sem0)
        pltpu.async_copy(chunk_src(1), buf1, sem1)

        def body(i, _):
            ch = i * 2
            pltpu.make_async_copy(chunk_src(0), buf0, sem0).wait()
            accumulate(buf0)

            @pl.when(ch + 2 < nch)
            def _():
                pltpu.async_copy(chunk_src(ch + 2), buf0, sem0)

            pltpu.make_async_copy(chunk_src(0), buf1, sem1).wait()
            accumulate(buf1)

            @pl.when(ch + 3 < nch)
            def _():
                pltpu.async_copy(chunk_src(ch + 3), buf1, sem1)

            return 0

        lax.fori_loop(0, nch // 2, body, 0)
        pltpu.sync_copy(acc, out_hbm.at[half, h])

    return _sc_partial


def _finalize_body(tc_ref, sc_ref, surv_ref, flat_ref, *, H, S, cpad,
                   num_windows, review_end, keep, tail_start):
    colsum = tc_ref[:, 0, :] + sc_ref[0] + sc_ref[1]

    col = lax.broadcasted_iota(jnp.int32, (cpad, num_windows), 0)
    wid = lax.broadcasted_iota(jnp.int32, (cpad, num_windows), 1)
    member = ((col >= _SINK) & (col < review_end)
              & ((col - _SINK) // _OMEGA == wid))
    mmat = member.astype(jnp.float32)
    ws = lax.dot_general(colsum, mmat,
                         dimension_numbers=(((1,), (0,)), ((), ())),
                         preferred_element_type=jnp.float32,
                         precision=lax.Precision.HIGHEST)

    widx = lax.broadcasted_iota(jnp.int32, (H, num_windows), 1)
    cur = ws
    picks = []
    for _ in range(_KWIN):
        mx = jnp.max(cur, axis=1, keepdims=True)
        sel = jnp.where(cur == mx, widx, num_windows)
        idx = jnp.min(sel, axis=1, keepdims=True)
        picks.append(idx)
        cur = jnp.where(widx == idx, -jnp.inf, cur)
    a, b, c = picks
    lo = jnp.minimum(jnp.minimum(a, b), c)
    hi = jnp.maximum(jnp.maximum(a, b), c)
    mid = a + b + c - lo - hi

    j = lax.broadcasted_iota(jnp.int32, (H, keep), 1)
    tj = j - _SINK
    wslot = tj // _OMEGA
    off = tj - wslot * _OMEGA
    wsel = jnp.where(wslot == 0, lo, jnp.where(wslot == 1, mid, hi))
    win_tok = wsel * _OMEGA + _SINK + off
    surv = jnp.where(j < _SINK, j,
                     jnp.where(j < tail_start, win_tok,
                               j - tail_start + review_end))
    surv_ref[...] = surv
    hrow = lax.broadcasted_iota(jnp.int32, (H, keep), 0)
    flat_ref[...] = surv + hrow * S


def _make_finalize(H, S, cpad, num_windows, review_end, keep, tail_start):
    body = functools.partial(_finalize_body, H=H, S=S, cpad=cpad,
                             num_windows=num_windows, review_end=review_end,
                             keep=keep, tail_start=tail_start)
    return pl.pallas_call(
        body,
        out_shape=[jax.ShapeDtypeStruct((H, keep), jnp.int32),
                   jax.ShapeDtypeStruct((H, keep), jnp.int32)],
    )


def _make_gather(total_rows, D, nw, nchunks, chunk):
    mesh = plsc.VectorSubcoreMesh(core_axis_name="c", subcore_axis_name="s")

    @functools.partial(
        pl.kernel,
        mesh=mesh,
        out_type=[jax.ShapeDtypeStruct((total_rows, D), jnp.float32),
                  jax.ShapeDtypeStruct((total_rows, D), jnp.float32)],
        scratch_types=[pltpu.VMEM((nchunks, chunk), jnp.int32),
                       pltpu.VMEM((2 * nchunks, chunk, D), jnp.float32),
                       pltpu.SemaphoreType.DMA,
                       pltpu.SemaphoreType.DMA],
    )
    def _gather(kt_hbm, vt_hbm, idx_hbm, ko_hbm, vo_hbm, idxv, rows, gsem,
                ssem):
        wid = lax.axis_index("s") * 2 + lax.axis_index("c")
        pltpu.sync_copy(idx_hbm.at[pl.ds(wid * nchunks, nchunks)], idxv)
        jobs = [(kt_hbm, ko_hbm, t) for t in range(nchunks)]
        jobs += [(vt_hbm, vo_hbm, t) for t in range(nchunks)]
        gathers = []
        for j, (src, _, t) in enumerate(jobs):
            gathers.append(pltpu.async_copy(src.at[idxv.at[t]], rows.at[j],
                                            gsem))
        for g in gathers:
            g.wait()
        stores = []
        for j, (_, dst, t) in enumerate(jobs):
            base = (wid * nchunks + t) * chunk
            stores.append(pltpu.async_copy(rows.at[j],
                                           dst.at[pl.ds(base, chunk)], ssem))
        for s in stores:
            s.wait()

    return _gather


def kernel(past_k, past_v, attn_score_cache):
    B, H, S, D = past_k.shape
    local_num = int(_PRATIO * S) // _OMEGA
    score_end = max(_SINK, S - local_num * _OMEGA - _OMEGA)
    num_windows = max(0, (score_end - _SINK) // _OMEGA)
    review_end = _SINK + num_windows * _OMEGA
    tail_start = _SINK + _KWIN * _OMEGA
    keep = tail_start + (S - review_end)

    attn3 = attn_score_cache[0]
    cpad = ((review_end + 127) // 128) * 128

    tc_part = _make_tc_partial(H, _S_TC, cpad)(attn3)
    sc_part = _make_sc_partial(H, S, _S_TC, cpad)(attn3)

    surv, flat = _make_finalize(H, S, cpad, num_windows, review_end, keep,
                                tail_start)(tc_part, sc_part)

    total_rows = H * keep
    nw = 32
    nchunks = 4
    chunk = total_rows // (nw * nchunks)

    idx2d = flat.reshape(nw * nchunks, chunk)
    kt = past_k[0].reshape(H * S, D)
    vt = past_v[0].reshape(H * S, D)

    ko, vo = _make_gather(total_rows, D, nw, nchunks, chunk)(kt, vt, idx2d)

    k_new = ko.reshape(1, H, keep, D)
    v_new = vo.reshape(1, H, keep, D)
    return k_new, v_new, surv


# speedup vs baseline: 1.7773x; 1.7773x over previous
import functools

import jax
import jax.numpy as jnp
from jax import lax
from jax.experimental import pallas as pl
from jax.experimental.pallas import tpu as pltpu
from jax.experimental.pallas import tpu_sc as plsc

_OMEGA = 64
_SINK = 5
_KWIN = 3
_PRATIO = 0.2

_S_TC = 1280
_RCH = 32
_LANES = 16


def _tc_partial_body(attn_ref, out_ref):
    out_ref[...] = jnp.sum(attn_ref[0], axis=0, keepdims=True)[None]


def _make_tc_partial(H, s_tc, cpad):
    return pl.pallas_call(
        _tc_partial_body,
        grid=(H,),
        in_specs=[pl.BlockSpec((1, s_tc, cpad), lambda h: (h, 0, 0))],
        out_specs=pl.BlockSpec((1, 1, cpad), lambda h: (h, 0, 0)),
        out_shape=jax.ShapeDtypeStruct((H, 1, cpad), jnp.float32),
    )


def _make_sc_partial(H, S, s_tc, cpad):
    rows_per_tile = (S - s_tc) // 2
    nch = rows_per_tile // _RCH
    ncs = cpad // _LANES
    mesh = plsc.VectorSubcoreMesh(core_axis_name="c", subcore_axis_name="s")

    @functools.partial(
        pl.kernel,
        mesh=mesh,
        out_type=jax.ShapeDtypeStruct((2, H, cpad), jnp.float32),
        scratch_types=[pltpu.VMEM((cpad,), jnp.float32),
                       pltpu.VMEM((_RCH, cpad), jnp.float32),
                       pltpu.VMEM((_RCH, cpad), jnp.float32),
                       pltpu.SemaphoreType.DMA,
                       pltpu.SemaphoreType.DMA],
    )
    def _sc_partial(attn_hbm, out_hbm, acc, buf0, buf1, sem0, sem1):
        h = lax.axis_index("s")
        half = lax.axis_index("c")
        r0 = s_tc + half * rows_per_tile

        def chunk_src(ch):
            return attn_hbm.at[h, pl.ds(r0 + ch * _RCH, _RCH),
                               pl.ds(0, cpad)]

        pltpu.async_copy(chunk_src(0), buf0, sem0)
        if nch > 1:
            pltpu.async_copy(chunk_src(1), buf1, sem1)

        for ch in range(nch):
            buf, sem = (buf0, sem0) if ch % 2 == 0 else (buf1, sem1)
            pltpu.make_async_copy(chunk_src(ch), buf, sem).wait()

            def cbody(c, _, buf=buf, first=(ch == 0)):
                sl = pl.ds(c * _LANES, _LANES)
                v0 = buf[0, sl]
                v1 = buf[1, sl]
                v2 = buf[2, sl]
                v3 = buf[3, sl]
                for r in range(4, _RCH, 4):
                    v0 = v0 + buf[r, sl]
                    v1 = v1 + buf[r + 1, sl]
                    v2 = v2 + buf[r + 2, sl]
                    v3 = v3 + buf[r + 3, sl]
                total = (v0 + v1) + (v2 + v3)
                if first:
                    acc[sl] = total
                else:
                    acc[sl] = acc[sl] + total
                return 0

            lax.fori_loop(0, ncs, cbody, 0)
            if ch + 2 < nch:
                pltpu.async_copy(chunk_src(ch + 2), buf, sem)

        pltpu.sync_copy(acc, out_hbm.at[half, h])

    return _sc_partial


def _finalize_body(tc_ref, sc_ref, surv_ref, flat_ref, *, H, S, cpad,
                   num_windows, review_end, keep, tail_start):
    colsum = tc_ref[:, 0, :] + sc_ref[0] + sc_ref[1]

    col = lax.broadcasted_iota(jnp.int32, (cpad, num_windows), 0)
    wid = lax.broadcasted_iota(jnp.int32, (cpad, num_windows), 1)
    member = ((col >= _SINK) & (col < review_end)
              & ((col - _SINK) // _OMEGA == wid))
    mmat = member.astype(jnp.float32)
    ws = lax.dot_general(colsum, mmat,
                         dimension_numbers=(((1,), (0,)), ((), ())),
                         preferred_element_type=jnp.float32,
                         precision=lax.Precision.HIGHEST)

    widx = lax.broadcasted_iota(jnp.int32, (H, num_windows), 1)
    cur = ws
    picks = []
    for _ in range(_KWIN):
        mx = jnp.max(cur, axis=1, keepdims=True)
        sel = jnp.where(cur == mx, widx, num_windows)
        idx = jnp.min(sel, axis=1, keepdims=True)
        picks.append(idx)
        cur = jnp.where(widx == idx, -jnp.inf, cur)
    a, b, c = picks
    lo = jnp.minimum(jnp.minimum(a, b), c)
    hi = jnp.maximum(jnp.maximum(a, b), c)
    mid = a + b + c - lo - hi

    j = lax.broadcasted_iota(jnp.int32, (H, keep), 1)
    tj = j - _SINK
    wslot = tj // _OMEGA
    off = tj - wslot * _OMEGA
    wsel = jnp.where(wslot == 0, lo, jnp.where(wslot == 1, mid, hi))
    win_tok = wsel * _OMEGA + _SINK + off
    surv = jnp.where(j < _SINK, j,
                     jnp.where(j < tail_start, win_tok,
                               j - tail_start + review_end))
    surv_ref[...] = surv
    hrow = lax.broadcasted_iota(jnp.int32, (H, keep), 0)
    flat_ref[...] = surv + hrow * S


def _make_finalize(H, S, cpad, num_windows, review_end, keep, tail_start):
    body = functools.partial(_finalize_body, H=H, S=S, cpad=cpad,
                             num_windows=num_windows, review_end=review_end,
                             keep=keep, tail_start=tail_start)
    return pl.pallas_call(
        body,
        out_shape=[jax.ShapeDtypeStruct((H, keep), jnp.int32),
                   jax.ShapeDtypeStruct((H, keep), jnp.int32)],
    )


def _make_gather(total_rows, D, nw, nchunks, chunk):
    mesh = plsc.VectorSubcoreMesh(core_axis_name="c", subcore_axis_name="s")

    @functools.partial(
        pl.kernel,
        mesh=mesh,
        out_type=[jax.ShapeDtypeStruct((total_rows, D), jnp.float32),
                  jax.ShapeDtypeStruct((total_rows, D), jnp.float32)],
        scratch_types=[pltpu.VMEM((nchunks, chunk), jnp.int32),
                       pltpu.VMEM((2 * nchunks, chunk, D), jnp.float32),
                       pltpu.SemaphoreType.DMA,
                       pltpu.SemaphoreType.DMA],
    )
    def _gather(kt_hbm, vt_hbm, idx_hbm, ko_hbm, vo_hbm, idxv, rows, gsem,
                ssem):
        wid = lax.axis_index("s") * 2 + lax.axis_index("c")
        pltpu.sync_copy(idx_hbm.at[pl.ds(wid * nchunks, nchunks)], idxv)
        jobs = [(kt_hbm, ko_hbm, t) for t in range(nchunks)]
        jobs += [(vt_hbm, vo_hbm, t) for t in range(nchunks)]
        gathers = []
        for j, (src, _, t) in enumerate(jobs):
            gathers.append(pltpu.async_copy(src.at[idxv.at[t]], rows.at[j],
                                            gsem))
        for g in gathers:
            g.wait()
        stores = []
        for j, (_, dst, t) in enumerate(jobs):
            base = (wid * nchunks + t) * chunk
            stores.append(pltpu.async_copy(rows.at[j],
                                           dst.at[pl.ds(base, chunk)], ssem))
        for s in stores:
            s.wait()

    return _gather


def kernel(past_k, past_v, attn_score_cache):
    B, H, S, D = past_k.shape
    local_num = int(_PRATIO * S) // _OMEGA
    score_end = max(_SINK, S - local_num * _OMEGA - _OMEGA)
    num_windows = max(0, (score_end - _SINK) // _OMEGA)
    review_end = _SINK + num_windows * _OMEGA
    tail_start = _SINK + _KWIN * _OMEGA
    keep = tail_start + (S - review_end)

    attn3 = attn_score_cache[0]
    cpad = ((review_end + 127) // 128) * 128

    tc_part = _make_tc_partial(H, _S_TC, cpad)(attn3)
    sc_part = _make_sc_partial(H, S, _S_TC, cpad)(attn3)

    surv, flat = _make_finalize(H, S, cpad, num_windows, review_end, keep,
                                tail_start)(tc_part, sc_part)

    total_rows = H * keep
    nw = 32
    nchunks = 4
    chunk = total_rows // (nw * nchunks)

    idx2d = flat.reshape(nw * nchunks, chunk)
    kt = past_k[0].reshape(H * S, D)
    vt = past_v[0].reshape(H * S, D)

    ko, vo = _make_gather(total_rows, D, nw, nchunks, chunk)(kt, vt, idx2d)

    k_new = ko.reshape(1, H, keep, D)
    v_new = vo.reshape(1, H, keep, D)
    return k_new, v_new, surv


# speedup vs baseline: 1.9174x; 1.0788x over previous
import functools

import jax
import jax.numpy as jnp
from jax import lax
from jax.experimental import pallas as pl
from jax.experimental.pallas import tpu as pltpu
from jax.experimental.pallas import tpu_sc as plsc

_OMEGA = 64
_SINK = 5
_KWIN = 3
_PRATIO = 0.2


def _score_body(attn_ref, surv_ref, flat_ref, acc_ref, *, nr, num_windows,
                review_end, S, keep, tail_start):
    h = pl.program_id(0)
    r = pl.program_id(1)

    block = attn_ref[0]
    chunk_sum = jnp.sum(block, axis=0, keepdims=True)

    @pl.when(r == 0)
    def _init():
        acc_ref[...] = chunk_sum

    @pl.when(r > 0)
    def _acc():
        acc_ref[...] += chunk_sum

    @pl.when(r == nr - 1)
    def _finish():
        colsum = acc_ref[...]
        cpad = colsum.shape[1]

        col = lax.broadcasted_iota(jnp.int32, (cpad, num_windows), 0)
        wid = lax.broadcasted_iota(jnp.int32, (cpad, num_windows), 1)
        member = ((col >= _SINK) & (col < review_end)
                  & ((col - _SINK) // _OMEGA == wid))
        mmat = member.astype(jnp.float32)
        ws = lax.dot_general(colsum, mmat,
                             dimension_numbers=(((1,), (0,)), ((), ())),
                             preferred_element_type=jnp.float32,
                             precision=lax.Precision.HIGHEST)

        widx = lax.broadcasted_iota(jnp.int32, (1, num_windows), 1)
        cur = ws
        picks = []
        for _ in range(_KWIN):
            mx = jnp.max(cur, axis=1, keepdims=True)
            sel = jnp.where(cur == mx, widx, num_windows)
            idx = jnp.min(sel, axis=1, keepdims=True)
            picks.append(idx)
            cur = jnp.where(widx == idx, -jnp.inf, cur)
        a, b, c = picks
        lo = jnp.minimum(jnp.minimum(a, b), c)
        hi = jnp.maximum(jnp.maximum(a, b), c)
        mid = a + b + c - lo - hi

        j = lax.broadcasted_iota(jnp.int32, (1, keep), 1)
        tj = j - _SINK
        wslot = tj // _OMEGA
        off = tj - wslot * _OMEGA
        wsel = jnp.where(wslot == 0, lo, jnp.where(wslot == 1, mid, hi))
        win_tok = wsel * _OMEGA + _SINK + off
        surv = jnp.where(j < _SINK, j,
                         jnp.where(j < tail_start, win_tok,
                                   j - tail_start + review_end))
        surv_ref[...] = surv[None]

        ncol = keep // 8
        j2 = (lax.broadcasted_iota(jnp.int32, (8, ncol), 0) * ncol
              + lax.broadcasted_iota(jnp.int32, (8, ncol), 1))
        tj2 = j2 - _SINK
        wslot2 = tj2 // _OMEGA
        off2 = tj2 - wslot2 * _OMEGA
        wsel2 = jnp.where(wslot2 == 0, lo, jnp.where(wslot2 == 1, mid, hi))
        win_tok2 = wsel2 * _OMEGA + _SINK + off2
        surv2 = jnp.where(j2 < _SINK, j2,
                          jnp.where(j2 < tail_start, win_tok2,
                                    j2 - tail_start + review_end))
        flat_ref[...] = surv2 + h * S


def _make_score_call(H, S, keep, nr, rchunk, cpad, num_windows, review_end,
                     tail_start):
    body = functools.partial(_score_body, nr=nr, num_windows=num_windows,
                             review_end=review_end, S=S, keep=keep,
                             tail_start=tail_start)
    return pl.pallas_call(
        body,
        grid=(H, nr),
        in_specs=[pl.BlockSpec((1, rchunk, cpad), lambda h, r: (h, r, 0))],
        out_specs=[pl.BlockSpec((1, 1, keep), lambda h, r: (h, 0, 0)),
                   pl.BlockSpec((8, keep // 8), lambda h, r: (h, 0))],
        out_shape=[jax.ShapeDtypeStruct((H, 1, keep), jnp.int32),
                   jax.ShapeDtypeStruct((H * 8, keep // 8), jnp.int32)],
        scratch_shapes=[pltpu.VMEM((1, cpad), jnp.float32)],
    )


def _make_gather(total_rows, D, nw, nchunks, chunk):
    mesh = plsc.VectorSubcoreMesh(core_axis_name="c", subcore_axis_name="s")

    @functools.partial(
        pl.kernel,
        mesh=mesh,
        out_type=[jax.ShapeDtypeStruct((total_rows, D), jnp.float32),
                  jax.ShapeDtypeStruct((total_rows, D), jnp.float32)],
        scratch_types=[pltpu.VMEM((nchunks, chunk), jnp.int32),
                       pltpu.VMEM((2 * nchunks, chunk, D), jnp.float32),
                       pltpu.SemaphoreType.DMA,
                       pltpu.SemaphoreType.DMA],
    )
    def _gather(kt_hbm, vt_hbm, idx_hbm, ko_hbm, vo_hbm, idxv, rows, gsem,
                ssem):
        wid = lax.axis_index("s") * 2 + lax.axis_index("c")
        pltpu.sync_copy(idx_hbm.at[pl.ds(wid * nchunks, nchunks)], idxv)
        jobs = [(kt_hbm, ko_hbm, t) for t in range(nchunks)]
        jobs += [(vt_hbm, vo_hbm, t) for t in range(nchunks)]
        gathers = []
        for j, (src, _, t) in enumerate(jobs):
            gathers.append(pltpu.async_copy(src.at[idxv.at[t]], rows.at[j],
                                            gsem))
        for g in gathers:
            g.wait()
        stores = []
        for j, (_, dst, t) in enumerate(jobs):
            base = (wid * nchunks + t) * chunk
            stores.append(pltpu.async_copy(rows.at[j],
                                           dst.at[pl.ds(base, chunk)], ssem))
        for s in stores:
            s.wait()

    return _gather


def kernel(past_k, past_v, attn_score_cache):
    B, H, S, D = past_k.shape
    local_num = int(_PRATIO * S) // _OMEGA
    score_end = max(_SINK, S - local_num * _OMEGA - _OMEGA)
    num_windows = max(0, (score_end - _SINK) // _OMEGA)
    review_end = _SINK + num_windows * _OMEGA
    tail_start = _SINK + _KWIN * _OMEGA
    keep = tail_start + (S - review_end)

    attn3 = attn_score_cache[0]
    cpad = ((review_end + 127) // 128) * 128
    rchunk = 2048
    nr = S // rchunk

    surv, flat = _make_score_call(H, S, keep, nr, rchunk, cpad, num_windows,
                                  review_end, tail_start)(attn3)
    surv = surv.reshape(H, keep)

    total_rows = H * keep
    nw = 32
    nchunks = 4
    chunk = total_rows // (nw * nchunks)

    idx2d = flat
    kt = past_k[0].reshape(H * S, D)
    vt = past_v[0].reshape(H * S, D)

    ko, vo = _make_gather(total_rows, D, nw, nchunks, chunk)(kt, vt, idx2d)

    k_new = ko.reshape(1, H, keep, D)
    v_new = vo.reshape(1, H, keep, D)
    return k_new, v_new, surv
